# Initial kernel scaffold; baseline (speedup 1.0000x reference)
import functools

import jax
import jax.numpy as jnp
from jax import lax
from jax.experimental import pallas as pl
from jax.experimental.pallas import tpu as pltpu
from jax.experimental.pallas import tpu_sc as plsc

N = 10000
E = 320000
H = 128
NUM_CODES = 16

NC = 2
NS = 16
NW = NC * NS
EDGES_PER_TILE = E // NW
CHUNK = 80
NCHUNK = EDGES_PER_TILE // CHUNK
ROWS_PER_TILE = N // NS
ZROWS = 125


def _prep_body(attr_ref, emb_ref, linw_ref, linb_ref, h_ref, e16_ref):
    a = attr_ref[...]
    h_ref[...] = (a[0:1, :] + 3 * a[1:2, :] + 7 * a[2:3, :]) & (NUM_CODES - 1)

    @pl.when(pl.program_id(0) == 0)
    def _():
        e16_ref[...] = (
            jnp.dot(emb_ref[...], linw_ref[...],
                    preferred_element_type=jnp.float32)
            + linb_ref[...]
        )


def _prep(attr_t, emb, lin_W, lin_b2d):
    eb = E // 10
    return pl.pallas_call(
        _prep_body,
        grid=(10,),
        in_specs=[
            pl.BlockSpec((3, eb), lambda i: (0, i)),
            pl.BlockSpec((NUM_CODES, 8), lambda i: (0, 0)),
            pl.BlockSpec((8, H), lambda i: (0, 0)),
            pl.BlockSpec((1, H), lambda i: (0, 0)),
        ],
        out_specs=[
            pl.BlockSpec((1, eb), lambda i: (0, i)),
            pl.BlockSpec((NUM_CODES, H), lambda i: (0, 0)),
        ],
        out_shape=[
            jax.ShapeDtypeStruct((1, E), jnp.int32),
            jax.ShapeDtypeStruct((NUM_CODES, H), jnp.float32),
        ],
    )(attr_t, emb, lin_W, lin_b2d)


def _sc_body(x_hbm, e16_hbm, src_hbm, dst_hbm, h_hbm, part_hbm,
             src_v, h_v, dst_v, xrows, erows, zeros_v, acc_sh, sem1, sem2):
    cid = lax.axis_index("c")
    sid = lax.axis_index("s")

    zv = jnp.zeros((16,), jnp.float32)

    def zero_body(j, carry):
        for g in range(8):
            zeros_v[j, pl.ds(g * 16, 16)] = zv
        return carry

    lax.fori_loop(0, ZROWS, zero_body, 0)
    for r in range(ROWS_PER_TILE // ZROWS):
        pltpu.sync_copy(zeros_v,
                        acc_sh.at[pl.ds((sid * 5 + r) * ZROWS, ZROWS)])
    plsc.subcore_barrier()

    base = (cid * NS + sid) * EDGES_PER_TILE

    def chunk_body(k, carry):
        off = base + k * CHUNK
        pltpu.sync_copy(src_hbm.at[pl.ds(off, CHUNK)], src_v)
        pltpu.sync_copy(h_hbm.at[pl.ds(off, CHUNK)], h_v)
        pltpu.sync_copy(dst_hbm.at[pl.ds(off, CHUNK)], dst_v)
        cp1 = pltpu.async_copy(x_hbm.at[src_v], xrows, sem1)
        cp2 = pltpu.async_copy(e16_hbm.at[h_v], erows, sem2)
        cp1.wait()
        cp2.wait()

        def msg_body(j, c):
            for g in range(8):
                s = pl.ds(g * 16, 16)
                xrows[j, s] = jnp.maximum(xrows[j, s] + erows[j, s], 0.0)
            return c

        lax.fori_loop(0, CHUNK, msg_body, 0)
        pltpu.sync_copy(xrows, acc_sh.at[dst_v], add=True)
        return carry

    lax.fori_loop(0, NCHUNK, chunk_body, 0)
    plsc.subcore_barrier()

    for r in range(ROWS_PER_TILE // ZROWS):
        rs = pl.ds((sid * 5 + r) * ZROWS, ZROWS)
        pltpu.sync_copy(acc_sh.at[rs], part_hbm.at[cid, rs])


def _sc_agg(x, e16, src, dst, h):
    mesh = plsc.VectorSubcoreMesh(core_axis_name="c", subcore_axis_name="s")
    f = pl.kernel(
        _sc_body,
        out_type=jax.ShapeDtypeStruct((NC, N, H), jnp.float32),
        mesh=mesh,
        scratch_types=[
            pltpu.VMEM((CHUNK,), jnp.int32),
            pltpu.VMEM((CHUNK,), jnp.int32),
            pltpu.VMEM((CHUNK,), jnp.int32),
            pltpu.VMEM((CHUNK, H), jnp.float32),
            pltpu.VMEM((CHUNK, H), jnp.float32),
            pltpu.VMEM((ZROWS, H), jnp.float32),
            pltpu.VMEM_SHARED((N, H), jnp.float32),
            pltpu.SemaphoreType.DMA,
            pltpu.SemaphoreType.DMA,
        ],
    )
    return f(x, e16, src, dst, h)


def _mlp_body(x_ref, part_ref, w1_ref, b1_ref, w2_ref, b2_ref, eps_ref,
              out_ref):
    z = ((1.0 + eps_ref[0, 0]) * x_ref[...]
         + part_ref[0, :, :] + part_ref[1, :, :])
    hh = jnp.maximum(
        jnp.dot(z, w1_ref[...], preferred_element_type=jnp.float32)
        + b1_ref[...], 0.0)
    out_ref[...] = (
        jnp.dot(hh, w2_ref[...], preferred_element_type=jnp.float32)
        + b2_ref[...])


def _mlp(x, part, W1, b1_2d, W2, b2_2d, eps_2d):
    nb = 2000
    return pl.pallas_call(
        _mlp_body,
        grid=(N // nb,),
        in_specs=[
            pl.BlockSpec((nb, H), lambda i: (i, 0)),
            pl.BlockSpec((NC, nb, H), lambda i: (0, i, 0)),
            pl.BlockSpec((H, H), lambda i: (0, 0)),
            pl.BlockSpec((1, H), lambda i: (0, 0)),
            pl.BlockSpec((H, H), lambda i: (0, 0)),
            pl.BlockSpec((1, H), lambda i: (0, 0)),
            pl.BlockSpec(memory_space=pltpu.SMEM),
        ],
        out_specs=pl.BlockSpec((nb, H), lambda i: (i, 0)),
        out_shape=jax.ShapeDtypeStruct((N, H), jnp.float32),
    )(x, part, W1, b1_2d, W2, b2_2d, eps_2d)


def kernel(x, edge_index, edge_attr, emb, lin_W, lin_b, W1, b1, W2, b2, eps):
    h2d, e16 = _prep(edge_attr.T, emb, lin_W, lin_b.reshape(1, H))
    src = edge_index[0]
    dst = edge_index[1]
    part = _sc_agg(x, e16, src, dst, h2d.reshape(E))
    return _mlp(x, part, W1, b1.reshape(1, H), W2, b2.reshape(1, H),
                eps.reshape(1, 1))


# speedup vs baseline: 2.9140x; 2.9140x over previous
import functools

import jax
import jax.numpy as jnp
from jax import lax
from jax.experimental import pallas as pl
from jax.experimental.pallas import tpu as pltpu
from jax.experimental.pallas import tpu_sc as plsc

N = 10000
E = 320000
H = 128
NUM_CODES = 16

NC = 2
NS = 16
NW = NC * NS
EDGES_PER_TILE = E // NW
CHUNK = 80
NCHUNK = EDGES_PER_TILE // CHUNK
N_PAD = 10240
ROWS_PER_TILE = N_PAD // NS
ZROWS = 128


def _prep_body(attr_ref, emb_ref, linw_ref, linb_ref, h_ref, e16_ref):
    a = attr_ref[...]
    h_ref[...] = (a[0:1, :] + 3 * a[1:2, :] + 7 * a[2:3, :]) & (NUM_CODES - 1)

    @pl.when(pl.program_id(0) == 0)
    def _():
        e16_ref[...] = (
            jnp.dot(emb_ref[...], linw_ref[...],
                    preferred_element_type=jnp.float32)
            + linb_ref[...]
        )


def _prep(attr_t, emb, lin_W, lin_b2d):
    eb = E // 10
    return pl.pallas_call(
        _prep_body,
        grid=(10,),
        in_specs=[
            pl.BlockSpec((3, eb), lambda i: (0, i)),
            pl.BlockSpec((NUM_CODES, 8), lambda i: (0, 0)),
            pl.BlockSpec((8, H), lambda i: (0, 0)),
            pl.BlockSpec((1, H), lambda i: (0, 0)),
        ],
        out_specs=[
            pl.BlockSpec((1, eb), lambda i: (0, i)),
            pl.BlockSpec((NUM_CODES, H), lambda i: (0, 0)),
        ],
        out_shape=[
            jax.ShapeDtypeStruct((1, E), jnp.int32),
            jax.ShapeDtypeStruct((NUM_CODES, H), jnp.float32),
        ],
    )(attr_t, emb, lin_W, lin_b2d)


def _sc_body(x_hbm, e16_hbm, src_hbm, dst_hbm, h_hbm, part_hbm,
             src_v, h_v, dst_v, xrows, erows, zeros_v, acc_sh, sem1, sem2):
    cid = lax.axis_index("c")
    sid = lax.axis_index("s")

    zv = jnp.zeros((16,), jnp.float32)

    def zero_body(j, carry):
        for g in range(8):
            zeros_v[j, pl.ds(g * 16, 16)] = zv
        return carry

    lax.fori_loop(0, ZROWS, zero_body, 0)
    for r in range(ROWS_PER_TILE // ZROWS):
        pltpu.sync_copy(zeros_v,
                        acc_sh.at[pl.ds((sid * 5 + r) * ZROWS, ZROWS)])
    plsc.subcore_barrier()

    base = (cid * NS + sid) * EDGES_PER_TILE

    def chunk_body(k, carry):
        off = base + k * CHUNK
        pltpu.sync_copy(src_hbm.at[pl.ds(off, CHUNK)], src_v)
        pltpu.sync_copy(h_hbm.at[pl.ds(off, CHUNK)], h_v)
        pltpu.sync_copy(dst_hbm.at[pl.ds(off, CHUNK)], dst_v)
        cp1 = pltpu.async_copy(x_hbm.at[src_v], xrows, sem1)
        cp2 = pltpu.async_copy(e16_hbm.at[h_v], erows, sem2)
        cp1.wait()
        cp2.wait()

        def msg_body(j, c):
            for g in range(8):
                s = pl.ds(g * 16, 16)
                xrows[j, s] = jnp.maximum(xrows[j, s] + erows[j, s], 0.0)
            return c

        lax.fori_loop(0, CHUNK, msg_body, 0)
        pltpu.sync_copy(xrows, acc_sh.at[dst_v], add=True)
        return carry

    lax.fori_loop(0, NCHUNK, chunk_body, 0)
    plsc.subcore_barrier()

    for r in range(ROWS_PER_TILE // ZROWS):
        rs = pl.ds((sid * 5 + r) * ZROWS, ZROWS)
        pltpu.sync_copy(acc_sh.at[rs], part_hbm.at[cid, rs])


def _sc_agg(x, e16, src, dst, h):
    mesh = plsc.VectorSubcoreMesh(core_axis_name="c", subcore_axis_name="s")
    f = pl.kernel(
        _sc_body,
        out_type=jax.ShapeDtypeStruct((NC, N_PAD, H), jnp.float32),
        mesh=mesh,
        scratch_types=[
            pltpu.VMEM((CHUNK,), jnp.int32),
            pltpu.VMEM((CHUNK,), jnp.int32),
            pltpu.VMEM((CHUNK,), jnp.int32),
            pltpu.VMEM((CHUNK, H), jnp.float32),
            pltpu.VMEM((CHUNK, H), jnp.float32),
            pltpu.VMEM((ZROWS, H), jnp.float32),
            pltpu.VMEM_SHARED((N_PAD, H), jnp.float32),
            pltpu.SemaphoreType.DMA,
            pltpu.SemaphoreType.DMA,
        ],
    )
    return f(x, e16, src, dst, h)


def _mlp_body(x_ref, part_ref, w1_ref, b1_ref, w2_ref, b2_ref, eps_ref,
              out_ref):
    z = ((1.0 + eps_ref[0, 0]) * x_ref[...]
         + part_ref[0, :, :] + part_ref[1, :, :])
    hh = jnp.maximum(
        jnp.dot(z, w1_ref[...], preferred_element_type=jnp.float32)
        + b1_ref[...], 0.0)
    out_ref[...] = (
        jnp.dot(hh, w2_ref[...], preferred_element_type=jnp.float32)
        + b2_ref[...])


def _mlp(x, part, W1, b1_2d, W2, b2_2d, eps_2d):
    nb = 2000
    return pl.pallas_call(
        _mlp_body,
        grid=(N // nb,),
        in_specs=[
            pl.BlockSpec((nb, H), lambda i: (i, 0)),
            pl.BlockSpec((NC, nb, H), lambda i: (0, i, 0)),
            pl.BlockSpec((H, H), lambda i: (0, 0)),
            pl.BlockSpec((1, H), lambda i: (0, 0)),
            pl.BlockSpec((H, H), lambda i: (0, 0)),
            pl.BlockSpec((1, H), lambda i: (0, 0)),
            pl.BlockSpec(memory_space=pltpu.SMEM),
        ],
        out_specs=pl.BlockSpec((nb, H), lambda i: (i, 0)),
        out_shape=jax.ShapeDtypeStruct((N, H), jnp.float32),
    )(x, part, W1, b1_2d, W2, b2_2d, eps_2d)


def kernel(x, edge_index, edge_attr, emb, lin_W, lin_b, W1, b1, W2, b2, eps):
    h2d, e16 = _prep(edge_attr.T, emb, lin_W, lin_b.reshape(1, H))
    src = edge_index[0]
    dst = edge_index[1]
    part = _sc_agg(x, e16, src, dst, h2d.reshape(E))
    return _mlp(x, part, W1, b1.reshape(1, H), W2, b2.reshape(1, H),
                eps.reshape(1, 1))


# speedup vs baseline: 4.6534x; 1.5969x over previous
import functools

import jax
import jax.numpy as jnp
from jax import lax
from jax.experimental import pallas as pl
from jax.experimental.pallas import tpu as pltpu
from jax.experimental.pallas import tpu_sc as plsc

N = 10000
E = 320000
H = 128
NUM_CODES = 16

NC = 2
NS = 16
NW = NC * NS
EDGES_PER_TILE = E // NW
CHUNK = 80
NCHUNK = EDGES_PER_TILE // CHUNK
NBUF = 4
NGRP = CHUNK // 16
N_PAD = 10240
ROWS_PER_TILE = N_PAD // NS
ZROWS = 128


def _prep_body(attr_ref, emb_ref, linw_ref, linb_ref, h_ref, e16_ref):
    a = attr_ref[...]
    h_ref[...] = (a[0:1, :] + 3 * a[1:2, :] + 7 * a[2:3, :]) & (NUM_CODES - 1)

    @pl.when(pl.program_id(0) == 0)
    def _():
        e16_ref[...] = (
            jnp.dot(emb_ref[...], linw_ref[...],
                    preferred_element_type=jnp.float32)
            + linb_ref[...]
        )


def _prep(attr_t, emb, lin_W, lin_b2d):
    eb = E // 10
    return pl.pallas_call(
        _prep_body,
        grid=(10,),
        in_specs=[
            pl.BlockSpec((3, eb), lambda i: (0, i)),
            pl.BlockSpec((NUM_CODES, 8), lambda i: (0, 0)),
            pl.BlockSpec((8, H), lambda i: (0, 0)),
            pl.BlockSpec((1, H), lambda i: (0, 0)),
        ],
        out_specs=[
            pl.BlockSpec((1, eb), lambda i: (0, i)),
            pl.BlockSpec((NUM_CODES, H), lambda i: (0, 0)),
        ],
        out_shape=[
            jax.ShapeDtypeStruct((1, E), jnp.int32),
            jax.ShapeDtypeStruct((NUM_CODES, H), jnp.float32),
        ],
    )(attr_t, emb, lin_W, lin_b2d)


def _sc_body(x_hbm, e16_hbm, src_hbm, dst_hbm, h_hbm, part_hbm,
             src_all, dst_all, h_all, e16_v, xb, acc_sh, sg, ss, si):
    cid = lax.axis_index("c")
    sid = lax.axis_index("s")
    wid = cid * NS + sid
    base = wid * EDGES_PER_TILE

    pltpu.sync_copy(e16_hbm, e16_v)

    zv = jnp.zeros((16,), jnp.float32)

    def zero_body(j, carry):
        for g in range(8):
            xb[0, j, pl.ds(g * 16, 16)] = zv
        return carry

    lax.fori_loop(0, CHUNK, zero_body, 0)
    for r in range(ROWS_PER_TILE // CHUNK):
        pltpu.sync_copy(xb.at[0],
                        acc_sh.at[pl.ds((sid * 8 + r) * CHUNK, CHUNK)])
    plsc.subcore_barrier()

    def fire_idx(k, b):
        off = base + k * CHUNK
        pltpu.async_copy(src_hbm.at[pl.ds(off, CHUNK)], src_all.at[b],
                         si.at[b])
        pltpu.async_copy(dst_hbm.at[pl.ds(off, CHUNK)], dst_all.at[b],
                         si.at[b])
        pltpu.async_copy(h_hbm.at[pl.ds(off, CHUNK)], h_all.at[b], si.at[b])

    def wait_idx(b):
        for _ in range(3):
            pltpu.make_async_copy(src_hbm.at[pl.ds(0, CHUNK)],
                                  src_all.at[b], si.at[b]).wait()

    def fire_gather(b):
        pltpu.async_copy(x_hbm.at[src_all.at[b]], xb.at[b], sg.at[b])

    def wait_gather(b):
        pltpu.make_async_copy(x_hbm.at[src_all.at[b]], xb.at[b],
                              sg.at[b]).wait()

    def fire_scatter(b):
        pltpu.async_copy(xb.at[b], acc_sh.at[dst_all.at[b]], ss.at[b],
                         add=True)

    def wait_scatter(b):
        pltpu.make_async_copy(xb.at[b], acc_sh.at[dst_all.at[b]],
                              ss.at[b]).wait()

    def compute(b):
        def msg_body(jg, c):
            hv = h_all[b, pl.ds(jg * 16, 16)]
            for e in range(16):
                hj = hv[e]
                j = jg * 16 + e
                for g in range(8):
                    sl = pl.ds(g * 16, 16)
                    xb[b, j, sl] = jnp.maximum(
                        xb[b, j, sl] + e16_v[hj, sl], 0.0)
            return c

        lax.fori_loop(0, NGRP, msg_body, 0)

    fire_idx(0, 0)
    fire_idx(1, 1)
    wait_idx(0)
    fire_gather(0)

    def step(k, carry):
        b = lax.rem(k, NBUF)
        b1 = lax.rem(k + 1, NBUF)
        b2 = lax.rem(k + 2, NBUF)
        wait_gather(b)

        @pl.when(k + 1 < NCHUNK)
        def _():
            wait_idx(b1)
            fire_gather(b1)

        compute(b)
        fire_scatter(b)

        @pl.when(k >= 2)
        def _():
            wait_scatter(b2)

        @pl.when(k + 2 < NCHUNK)
        def _():
            fire_idx(k + 2, b2)

        return carry

    lax.fori_loop(0, NCHUNK, step, 0)
    wait_scatter((NCHUNK - 2) % NBUF)
    wait_scatter((NCHUNK - 1) % NBUF)
    plsc.subcore_barrier()

    for r in range(ROWS_PER_TILE // ZROWS):
        rs = pl.ds((sid * 5 + r) * ZROWS, ZROWS)
        pltpu.sync_copy(acc_sh.at[rs], part_hbm.at[cid, rs])


def _sc_agg(x, e16, src, dst, h):
    mesh = plsc.VectorSubcoreMesh(core_axis_name="c", subcore_axis_name="s")
    f = pl.kernel(
        _sc_body,
        out_type=jax.ShapeDtypeStruct((NC, N_PAD, H), jnp.float32),
        mesh=mesh,
        scratch_types=[
            pltpu.VMEM((NBUF, CHUNK), jnp.int32),
            pltpu.VMEM((NBUF, CHUNK), jnp.int32),
            pltpu.VMEM((NBUF, CHUNK), jnp.int32),
            pltpu.VMEM((NUM_CODES, H), jnp.float32),
            pltpu.VMEM((NBUF, CHUNK, H), jnp.float32),
            pltpu.VMEM_SHARED((N_PAD, H), jnp.float32),
            pltpu.SemaphoreType.DMA((NBUF,)),
            pltpu.SemaphoreType.DMA((NBUF,)),
            pltpu.SemaphoreType.DMA((NBUF,)),
        ],
    )
    return f(x, e16, src, dst, h)


def _mlp_body(x_ref, part_ref, w1_ref, b1_ref, w2_ref, b2_ref, eps_ref,
              out_ref):
    z = ((1.0 + eps_ref[0, 0]) * x_ref[...]
         + part_ref[0, :, :] + part_ref[1, :, :])
    hh = jnp.maximum(
        jnp.dot(z, w1_ref[...], preferred_element_type=jnp.float32)
        + b1_ref[...], 0.0)
    out_ref[...] = (
        jnp.dot(hh, w2_ref[...], preferred_element_type=jnp.float32)
        + b2_ref[...])


def _mlp(x, part, W1, b1_2d, W2, b2_2d, eps_2d):
    nb = 2000
    return pl.pallas_call(
        _mlp_body,
        grid=(N // nb,),
        in_specs=[
            pl.BlockSpec((nb, H), lambda i: (i, 0)),
            pl.BlockSpec((NC, nb, H), lambda i: (0, i, 0)),
            pl.BlockSpec((H, H), lambda i: (0, 0)),
            pl.BlockSpec((1, H), lambda i: (0, 0)),
            pl.BlockSpec((H, H), lambda i: (0, 0)),
            pl.BlockSpec((1, H), lambda i: (0, 0)),
            pl.BlockSpec(memory_space=pltpu.SMEM),
        ],
        out_specs=pl.BlockSpec((nb, H), lambda i: (i, 0)),
        out_shape=jax.ShapeDtypeStruct((N, H), jnp.float32),
    )(x, part, W1, b1_2d, W2, b2_2d, eps_2d)


def kernel(x, edge_index, edge_attr, emb, lin_W, lin_b, W1, b1, W2, b2, eps):
    h2d, e16 = _prep(edge_attr.T, emb, lin_W, lin_b.reshape(1, H))
    part = _sc_agg(x, e16, edge_index[0], edge_index[1], h2d.reshape(E))
    return _mlp(x, part, W1, b1.reshape(1, H), W2, b2.reshape(1, H),
                eps.reshape(1, 1))


# speedup vs baseline: 11.9284x; 2.5634x over previous
import functools

import jax
import jax.numpy as jnp
from jax import lax
from jax.experimental import pallas as pl
from jax.experimental.pallas import tpu as pltpu
from jax.experimental.pallas import tpu_sc as plsc

N = 10000
E = 320000
H = 128
NUM_CODES = 16

NC = 2
NS = 16
NW = NC * NS
EDGES_PER_TILE = E // NW
CHUNK = 80
NCHUNK = EDGES_PER_TILE // CHUNK
NBUF = 4
NGRP = CHUNK // 16
N_PAD = 10240
ROWS_PER_TILE = N_PAD // NS
ZROWS = 128


def _prep_body(attr_ref, emb_ref, linw_ref, linb_ref, h_ref, e16_ref):
    a = attr_ref[...]
    h_ref[...] = (a[0:1, :] + 3 * a[1:2, :] + 7 * a[2:3, :]) & (NUM_CODES - 1)

    @pl.when(pl.program_id(0) == 0)
    def _():
        e16_ref[...] = (
            jnp.dot(emb_ref[...], linw_ref[...],
                    preferred_element_type=jnp.float32)
            + linb_ref[...]
        )


def _prep(attr_t, emb, lin_W, lin_b2d):
    eb = E // 10
    return pl.pallas_call(
        _prep_body,
        grid=(10,),
        in_specs=[
            pl.BlockSpec((3, eb), lambda i: (0, i)),
            pl.BlockSpec((NUM_CODES, 8), lambda i: (0, 0)),
            pl.BlockSpec((8, H), lambda i: (0, 0)),
            pl.BlockSpec((1, H), lambda i: (0, 0)),
        ],
        out_specs=[
            pl.BlockSpec((1, eb), lambda i: (0, i)),
            pl.BlockSpec((NUM_CODES, H), lambda i: (0, 0)),
        ],
        out_shape=[
            jax.ShapeDtypeStruct((1, E), jnp.int32),
            jax.ShapeDtypeStruct((NUM_CODES, H), jnp.float32),
        ],
    )(attr_t, emb, lin_W, lin_b2d)


def _sc_body(x_hbm, e16_hbm, src_hbm, dst_hbm, h_hbm, part_hbm,
             src_all, dst_all, h_all, e16_v, xb, acc_sh, sg, ss, si):
    cid = lax.axis_index("c")
    sid = lax.axis_index("s")
    wid = cid * NS + sid
    base = wid * EDGES_PER_TILE

    pltpu.sync_copy(e16_hbm, e16_v)

    zv = jnp.zeros((16,), jnp.float32)

    @plsc.parallel_loop(0, CHUNK)
    def zero_body(j):
        for g in range(8):
            xb[j, pl.ds(g * 16, 16)] = zv

    for r in range(ROWS_PER_TILE // CHUNK):
        pltpu.sync_copy(xb.at[pl.ds(0, CHUNK)],
                        acc_sh.at[pl.ds((sid * 8 + r) * CHUNK, CHUNK)])
    plsc.subcore_barrier()

    def fire_idx(k, b):
        off = base + k * CHUNK
        pltpu.async_copy(src_hbm.at[pl.ds(off, CHUNK)], src_all.at[b],
                         si.at[b])
        pltpu.async_copy(dst_hbm.at[pl.ds(off, CHUNK)], dst_all.at[b],
                         si.at[b])
        pltpu.async_copy(h_hbm.at[pl.ds(off, CHUNK)], h_all.at[b], si.at[b])

    def wait_idx(b):
        for _ in range(3):
            pltpu.make_async_copy(src_hbm.at[pl.ds(0, CHUNK)],
                                  src_all.at[b], si.at[b]).wait()

    def slot(b):
        return pl.ds(pl.multiple_of(b * CHUNK, CHUNK), CHUNK)

    def fire_gather(b):
        pltpu.async_copy(x_hbm.at[src_all.at[b]], xb.at[slot(b)], sg.at[b])

    def wait_gather(b):
        pltpu.make_async_copy(x_hbm.at[src_all.at[b]], xb.at[slot(b)],
                              sg.at[b]).wait()

    def fire_scatter(b):
        pltpu.async_copy(xb.at[slot(b)], acc_sh.at[dst_all.at[b]], ss.at[b],
                         add=True)

    def wait_scatter(b):
        pltpu.make_async_copy(xb.at[slot(b)], acc_sh.at[dst_all.at[b]],
                              ss.at[b]).wait()

    def compute(b):
        rowbase = b * CHUNK

        @plsc.parallel_loop(0, NGRP)
        def msg_body(jg):
            hv = h_all[b, pl.ds(jg * 16, 16)]
            hjs = [hv[e] for e in range(16)]
            for e in range(16):
                row = rowbase + jg * 16 + e
                xs = [xb[row, pl.ds(g * 16, 16)] for g in range(8)]
                es = [e16_v[hjs[e], pl.ds(g * 16, 16)] for g in range(8)]
                for g in range(8):
                    xb[row, pl.ds(g * 16, 16)] = jnp.maximum(
                        xs[g] + es[g], 0.0)

    fire_idx(0, 0)
    fire_idx(1, 1)
    wait_idx(0)
    fire_gather(0)

    def step(k, carry):
        b = lax.rem(k, NBUF)
        b1 = lax.rem(k + 1, NBUF)
        b2 = lax.rem(k + 2, NBUF)
        wait_gather(b)

        @pl.when(k + 1 < NCHUNK)
        def _():
            wait_idx(b1)
            fire_gather(b1)

        compute(b)
        fire_scatter(b)

        @pl.when(k >= 2)
        def _():
            wait_scatter(b2)

        @pl.when(k + 2 < NCHUNK)
        def _():
            fire_idx(k + 2, b2)

        return carry

    lax.fori_loop(0, NCHUNK, step, 0)
    wait_scatter((NCHUNK - 2) % NBUF)
    wait_scatter((NCHUNK - 1) % NBUF)
    plsc.subcore_barrier()

    for r in range(ROWS_PER_TILE // ZROWS):
        rs = pl.ds((sid * 5 + r) * ZROWS, ZROWS)
        pltpu.sync_copy(acc_sh.at[rs], part_hbm.at[cid, rs])


def _sc_agg(x, e16, src, dst, h):
    mesh = plsc.VectorSubcoreMesh(core_axis_name="c", subcore_axis_name="s")
    f = pl.kernel(
        _sc_body,
        out_type=jax.ShapeDtypeStruct((NC, N_PAD, H), jnp.float32),
        mesh=mesh,
        scratch_types=[
            pltpu.VMEM((NBUF, CHUNK), jnp.int32),
            pltpu.VMEM((NBUF, CHUNK), jnp.int32),
            pltpu.VMEM((NBUF, CHUNK), jnp.int32),
            pltpu.VMEM((NUM_CODES, H), jnp.float32),
            pltpu.VMEM((NBUF * CHUNK, H), jnp.float32),
            pltpu.VMEM_SHARED((N_PAD, H), jnp.float32),
            pltpu.SemaphoreType.DMA((NBUF,)),
            pltpu.SemaphoreType.DMA((NBUF,)),
            pltpu.SemaphoreType.DMA((NBUF,)),
        ],
    )
    return f(x, e16, src, dst, h)


def _mlp_body(x_ref, part_ref, w1_ref, b1_ref, w2_ref, b2_ref, eps_ref,
              out_ref):
    z = ((1.0 + eps_ref[0, 0]) * x_ref[...]
         + part_ref[0, :, :] + part_ref[1, :, :])
    hh = jnp.maximum(
        jnp.dot(z, w1_ref[...], preferred_element_type=jnp.float32)
        + b1_ref[...], 0.0)
    out_ref[...] = (
        jnp.dot(hh, w2_ref[...], preferred_element_type=jnp.float32)
        + b2_ref[...])


def _mlp(x, part, W1, b1_2d, W2, b2_2d, eps_2d):
    nb = 2000
    return pl.pallas_call(
        _mlp_body,
        grid=(N // nb,),
        in_specs=[
            pl.BlockSpec((nb, H), lambda i: (i, 0)),
            pl.BlockSpec((NC, nb, H), lambda i: (0, i, 0)),
            pl.BlockSpec((H, H), lambda i: (0, 0)),
            pl.BlockSpec((1, H), lambda i: (0, 0)),
            pl.BlockSpec((H, H), lambda i: (0, 0)),
            pl.BlockSpec((1, H), lambda i: (0, 0)),
            pl.BlockSpec(memory_space=pltpu.SMEM),
        ],
        out_specs=pl.BlockSpec((nb, H), lambda i: (i, 0)),
        out_shape=jax.ShapeDtypeStruct((N, H), jnp.float32),
    )(x, part, W1, b1_2d, W2, b2_2d, eps_2d)


def kernel(x, edge_index, edge_attr, emb, lin_W, lin_b, W1, b1, W2, b2, eps):
    h2d, e16 = _prep(edge_attr.T, emb, lin_W, lin_b.reshape(1, H))
    part = _sc_agg(x, e16, edge_index[0], edge_index[1], h2d.reshape(E))
    return _mlp(x, part, W1, b1.reshape(1, H), W2, b2.reshape(1, H),
                eps.reshape(1, 1))
